# Initial kernel scaffold; baseline (speedup 1.0000x reference)
import jax
import jax.numpy as jnp
from jax.experimental import pallas as pl


def kernel(x, edge_index, batch, W, att_src, att_dst, bias):
    raise NotImplementedError("write your pallas kernel here")


# speedup vs baseline: 42.9404x; 42.9404x over previous
import functools

import jax
import jax.numpy as jnp
from jax import lax
from jax.experimental import pallas as pl
from jax.experimental.pallas import tpu as pltpu
from jax.experimental.pallas import tpu_sc as plsc

N = 10000
E = 320000
INF = 128
H = 4
C = 32
NG = 64
HC = H * C
HH = H // 2
HW = HH * C

NC = 2
NS = 16
E_TEC = E // NS
CHUNK = 80
NCHUNK = E_TEC // CHUNK
GROUPS = CHUNK // 16
ZROWS = 80
NWB = N // ZROWS
WB_TEC = -(-NWB // NS)

BLK = 1000
NBLK = N // BLK


def _proj_body(x_ref, w_ref, a2_ref, b2_ref,
               h0_ref, h1_ref, a_ref, amax_ref, cnt_ref):
    h = jnp.dot(x_ref[...], w_ref[...], preferred_element_type=jnp.float32)
    h0_ref[...] = h[:, :HW]
    h1_ref[...] = h[:, HW:]
    a = jnp.dot(h, a2_ref[...], preferred_element_type=jnp.float32,
                precision=lax.Precision.HIGHEST)
    a_ref[...] = a
    bm = jnp.max(a[:, :H], axis=0)
    bm16 = jnp.concatenate([bm, jnp.full((16 - H,), -jnp.inf, jnp.float32)])
    b = b2_ref[...]
    onehot = (b == lax.broadcasted_iota(jnp.int32, (BLK, NG), 1)).astype(jnp.float32)
    cnt = jnp.sum(onehot, axis=0, keepdims=True)

    @pl.when(pl.program_id(0) == 0)
    def _():
        amax_ref[...] = jnp.full((1, 16), -jnp.inf, jnp.float32)
        cnt_ref[...] = jnp.zeros((1, NG), jnp.float32)

    amax_ref[...] = jnp.maximum(amax_ref[...], bm16[None, :])
    cnt_ref[...] = cnt_ref[...] + cnt


_proj_call = pl.pallas_call(
    _proj_body,
    grid=(NBLK,),
    in_specs=[
        pl.BlockSpec((BLK, INF), lambda i: (i, 0)),
        pl.BlockSpec((INF, HC), lambda i: (0, 0)),
        pl.BlockSpec((HC, 2 * H), lambda i: (0, 0)),
        pl.BlockSpec((BLK, 1), lambda i: (i, 0)),
    ],
    out_specs=[
        pl.BlockSpec((BLK, HW), lambda i: (i, 0)),
        pl.BlockSpec((BLK, HW), lambda i: (i, 0)),
        pl.BlockSpec((BLK, 2 * H), lambda i: (i, 0)),
        pl.BlockSpec((1, 16), lambda i: (0, 0)),
        pl.BlockSpec((1, NG), lambda i: (0, 0)),
    ],
    out_shape=[
        jax.ShapeDtypeStruct((N, HW), jnp.float32),
        jax.ShapeDtypeStruct((N, HW), jnp.float32),
        jax.ShapeDtypeStruct((N, 2 * H), jnp.float32),
        jax.ShapeDtypeStruct((1, 16), jnp.float32),
        jax.ShapeDtypeStruct((1, NG), jnp.float32),
    ],
)


def _sc_body(h0_hbm, h1_hbm, a0_hbm, a1_hbm, amax_hbm, src_hbm, dst_hbm,
             zrow_hbm, zden_hbm,
             out_hbm, den_hbm,
             a_v, src_v, dst_v, rows_v, p0_v, p1_v, di0_v, di1_v,
             amax_v, zrow_v, zden_v,
             out_sh, den_sh,
             sem_rows, sem_den, sem_out):
    c = lax.axis_index("c")
    s = lax.axis_index("s")
    @pl.when(c == 0)
    def _():
        pltpu.sync_copy(a0_hbm, a_v)

    @pl.when(c != 0)
    def _():
        pltpu.sync_copy(a1_hbm, a_v)

    pltpu.sync_copy(amax_hbm, amax_v)
    pltpu.sync_copy(zrow_hbm, zrow_v)
    pltpu.sync_copy(zden_hbm, zden_v)
    for i in range(WB_TEC):
        j = s * WB_TEC + i

        @pl.when(j < NWB)
        def _():
            r0 = pl.multiple_of(j * ZROWS, 8)
            pltpu.sync_copy(zrow_v, out_sh.at[pl.ds(r0, ZROWS)])
            pltpu.sync_copy(zden_v, den_sh.at[pl.ds(r0 * HH, ZROWS * HH)])

    plsc.subcore_barrier()

    iota16 = lax.iota(jnp.int32, 16)
    base = s * E_TEC
    hbase = c * HH

    def chunk(ci, carry):
        eb = pl.multiple_of(base + ci * CHUNK, 8)
        pltpu.sync_copy(src_hbm.at[pl.ds(eb, CHUNK)], src_v)
        pltpu.sync_copy(dst_hbm.at[pl.ds(eb, CHUNK)], dst_v)

        @pl.when(c == 0)
        def _():
            pltpu.async_copy(h0_hbm.at[src_v], rows_v, sem_rows).wait()

        @pl.when(c != 0)
        def _():
            pltpu.async_copy(h1_hbm.at[src_v], rows_v, sem_rows).wait()

        for g in range(GROUPS):
            src16 = src_v[pl.ds(g * 16, 16)]
            dst16 = dst_v[pl.ds(g * 16, 16)]
            for hh in range(HH):
                asrc = plsc.load_gather(a_v, [src16 * 4 + hh])
                adst = plsc.load_gather(a_v, [dst16 * 4 + (2 + hh)])
                al = asrc + adst
                al = jnp.where(al >= 0.0, al, 0.2 * al)
                amx = plsc.load_gather(amax_v, [iota16 * 0 + (hbase + hh)])
                bnd = amx + adst
                bnd = jnp.where(bnd >= 0.0, bnd, 0.2 * bnd)
                p16 = jnp.exp(al - bnd)
                pv = p0_v if hh == 0 else p1_v
                div = di0_v if hh == 0 else di1_v
                pv[pl.ds(16 + g * 16, 16)] = p16
                div[pl.ds(g * 16, 16)] = dst16 * HH + hh
        pltpu.sync_copy(p0_v.at[pl.ds(16, CHUNK)], den_sh.at[di0_v], add=True)
        pltpu.sync_copy(p1_v.at[pl.ds(16, CHUNK)], den_sh.at[di1_v], add=True)
        for g in range(GROUPS):
            for hh in range(HH):
                pv = p0_v if hh == 0 else p1_v
                for j in range(16):
                    e = g * 16 + j
                    w = plsc.load_gather(pv, [iota16 * 0 + (16 + e)])
                    rows_v[e, pl.ds(hh * C, 16)] = rows_v[e, pl.ds(hh * C, 16)] * w
                    rows_v[e, pl.ds(hh * C + 16, 16)] = (
                        rows_v[e, pl.ds(hh * C + 16, 16)] * w)
        pltpu.sync_copy(rows_v, out_sh.at[dst_v], add=True)
        return carry

    lax.fori_loop(0, NCHUNK, chunk, 0)
    plsc.subcore_barrier()
    for i in range(WB_TEC):
        j = s * WB_TEC + i

        @pl.when(j < NWB)
        def _():
            r0 = pl.multiple_of(j * ZROWS, 8)
            pltpu.sync_copy(out_sh.at[pl.ds(r0, ZROWS)], zrow_v)
            pltpu.sync_copy(zrow_v, out_hbm.at[c, pl.ds(r0, ZROWS)])
            pltpu.sync_copy(den_sh.at[pl.ds(r0 * HH, ZROWS * HH)], zden_v)
            d0 = pl.multiple_of(c * (N * HH) + r0 * HH, 8)
            pltpu.sync_copy(zden_v, den_hbm.at[pl.ds(d0, ZROWS * HH)])


@functools.lru_cache(maxsize=1)
def _get_sc_call():
    mesh = plsc.VectorSubcoreMesh(
        core_axis_name="c", subcore_axis_name="s",
        num_cores=NC, num_subcores=NS)
    return pl.kernel(
        _sc_body,
        out_type=[
            jax.ShapeDtypeStruct((NC, N, HW), jnp.float32),
            jax.ShapeDtypeStruct((NC * N * HH,), jnp.float32),
        ],
        mesh=mesh,
        compiler_params=pltpu.CompilerParams(
            needs_layout_passes=False, use_tc_tiling_on_sc=False),
        scratch_types=[
            pltpu.VMEM((N * 4,), jnp.float32),
            pltpu.VMEM((CHUNK,), jnp.int32),
            pltpu.VMEM((CHUNK,), jnp.int32),
            pltpu.VMEM((CHUNK, HW), jnp.float32),
            pltpu.VMEM((16 + CHUNK,), jnp.float32),
            pltpu.VMEM((16 + CHUNK,), jnp.float32),
            pltpu.VMEM((CHUNK,), jnp.int32),
            pltpu.VMEM((CHUNK,), jnp.int32),
            pltpu.VMEM((16,), jnp.float32),
            pltpu.VMEM((ZROWS, HW), jnp.float32),
            pltpu.VMEM((ZROWS * HH,), jnp.float32),
            pltpu.VMEM_SHARED((N, HW), jnp.float32),
            pltpu.VMEM_SHARED((N * HH,), jnp.float32),
            pltpu.SemaphoreType.DMA,
            pltpu.SemaphoreType.DMA,
            pltpu.SemaphoreType.DMA,
        ],
    )


def _fin_body(acc_ref, den_ref, bias_ref, cnt_ref, b2_ref, exp_ref, out_ref):
    inv = lax.rsqrt(jnp.maximum(cnt_ref[...], 1.0))
    b = b2_ref[...]
    onehot = (b == lax.broadcasted_iota(jnp.int32, (BLK, NG), 1)).astype(jnp.float32)
    gamma = jnp.sum(onehot * inv, axis=1, keepdims=True)
    for c in range(NC):
        acc = acc_ref[c]
        den = den_ref[c] + 1e-16
        denx = jnp.dot(den, exp_ref[...], preferred_element_type=jnp.float32)
        o = acc / denx + bias_ref[:, c * HW:(c + 1) * HW]
        o = jnp.maximum(o, 0.0)
        out_ref[:, c * HW:(c + 1) * HW] = o * gamma


_fin_call = pl.pallas_call(
    _fin_body,
    grid=(NBLK,),
    in_specs=[
        pl.BlockSpec((NC, BLK, HW), lambda i: (0, i, 0)),
        pl.BlockSpec((NC, BLK, HH), lambda i: (0, i, 0)),
        pl.BlockSpec((1, HC), lambda i: (0, 0)),
        pl.BlockSpec((1, NG), lambda i: (0, 0)),
        pl.BlockSpec((BLK, 1), lambda i: (i, 0)),
        pl.BlockSpec((HH, HW), lambda i: (0, 0)),
    ],
    out_specs=pl.BlockSpec((BLK, HC), lambda i: (i, 0)),
    out_shape=jax.ShapeDtypeStruct((N, HC), jnp.float32),
)


def kernel(x, edge_index, batch, W, att_src, att_dst, bias):
    eyeH = jnp.eye(H, dtype=jnp.float32)
    a_s = (att_src[:, :, None] * eyeH[:, None, :]).reshape(HC, H)
    a_d = (att_dst[:, :, None] * eyeH[:, None, :]).reshape(HC, H)
    a2 = jnp.concatenate([a_s, a_d], axis=1)
    batch2 = batch.reshape(N, 1)

    h0, h1, a, amax, counts = _proj_call(x, W, a2, batch2)

    a0 = a[:, (0, 1, H, H + 1)].reshape(N * 4)
    a1 = a[:, (2, 3, H + 2, H + 3)].reshape(N * 4)
    src = edge_index[0]
    dst = edge_index[1]
    zrow = jnp.zeros((ZROWS, HW), jnp.float32)
    zden = jnp.zeros((ZROWS * HH,), jnp.float32)
    out2, den2 = _get_sc_call()(
        h0, h1, a0, a1, amax.reshape(16), src, dst, zrow, zden)

    expand = jnp.kron(jnp.eye(HH, dtype=jnp.float32), jnp.ones((1, C), jnp.float32))
    return _fin_call(out2, den2.reshape(NC, N, HH), bias.reshape(1, HC),
                     counts, batch2, expand)


# speedup vs baseline: 47.5326x; 1.1069x over previous
import functools

import jax
import jax.numpy as jnp
from jax import lax
from jax.experimental import pallas as pl
from jax.experimental.pallas import tpu as pltpu
from jax.experimental.pallas import tpu_sc as plsc

N = 10000
E = 320000
INF = 128
H = 4
C = 32
NG = 64
HC = H * C
HH = H // 2
HW = HH * C

NC = 2
NS = 16
E_TEC = E // NS
CHUNK = 80
NCHUNK = E_TEC // CHUNK
GROUPS = CHUNK // 16
ZROWS = 80
NWB = N // ZROWS
WB_TEC = -(-NWB // NS)

BLK = 1000
NBLK = N // BLK


def _proj_body(x_ref, w_ref, a2_ref, b2_ref,
               h0_ref, h1_ref, a_ref, amax_ref, cnt_ref):
    h = jnp.dot(x_ref[...], w_ref[...], preferred_element_type=jnp.float32)
    h0_ref[...] = h[:, :HW]
    h1_ref[...] = h[:, HW:]
    a = jnp.dot(h, a2_ref[...], preferred_element_type=jnp.float32,
                precision=lax.Precision.HIGHEST)
    a_ref[...] = a
    bm = jnp.max(a[:, :H], axis=0)
    bm16 = jnp.concatenate([bm, jnp.full((16 - H,), -jnp.inf, jnp.float32)])
    b = b2_ref[...]
    onehot = (b == lax.broadcasted_iota(jnp.int32, (BLK, NG), 1)).astype(jnp.float32)
    cnt = jnp.sum(onehot, axis=0, keepdims=True)

    @pl.when(pl.program_id(0) == 0)
    def _():
        amax_ref[...] = jnp.full((1, 16), -jnp.inf, jnp.float32)
        cnt_ref[...] = jnp.zeros((1, NG), jnp.float32)

    amax_ref[...] = jnp.maximum(amax_ref[...], bm16[None, :])
    cnt_ref[...] = cnt_ref[...] + cnt


_proj_call = pl.pallas_call(
    _proj_body,
    grid=(NBLK,),
    in_specs=[
        pl.BlockSpec((BLK, INF), lambda i: (i, 0)),
        pl.BlockSpec((INF, HC), lambda i: (0, 0)),
        pl.BlockSpec((HC, 2 * H), lambda i: (0, 0)),
        pl.BlockSpec((BLK, 1), lambda i: (i, 0)),
    ],
    out_specs=[
        pl.BlockSpec((BLK, HW), lambda i: (i, 0)),
        pl.BlockSpec((BLK, HW), lambda i: (i, 0)),
        pl.BlockSpec((BLK, 2 * H), lambda i: (i, 0)),
        pl.BlockSpec((1, 16), lambda i: (0, 0)),
        pl.BlockSpec((1, NG), lambda i: (0, 0)),
    ],
    out_shape=[
        jax.ShapeDtypeStruct((N, HW), jnp.float32),
        jax.ShapeDtypeStruct((N, HW), jnp.float32),
        jax.ShapeDtypeStruct((N, 2 * H), jnp.float32),
        jax.ShapeDtypeStruct((1, 16), jnp.float32),
        jax.ShapeDtypeStruct((1, NG), jnp.float32),
    ],
)


def _sc_body(h0_hbm, h1_hbm, a0_hbm, a1_hbm, amax_hbm, src_hbm, dst_hbm,
             zrow_hbm, zden_hbm,
             out_hbm, den_hbm,
             a_v, src_v, dst_v, rows_v, p0_v, p1_v, di0_v, di1_v,
             amax_v, zrow_v, zden_v,
             out_sh, den_sh,
             sem_rows, sem_den, sem_out):
    c = lax.axis_index("c")
    s = lax.axis_index("s")
    @pl.when(c == 0)
    def _():
        pltpu.sync_copy(a0_hbm, a_v)

    @pl.when(c != 0)
    def _():
        pltpu.sync_copy(a1_hbm, a_v)

    pltpu.sync_copy(amax_hbm, amax_v)
    pltpu.sync_copy(zrow_hbm, zrow_v)
    pltpu.sync_copy(zden_hbm, zden_v)
    for i in range(WB_TEC):
        j = s * WB_TEC + i

        @pl.when(j < NWB)
        def _():
            r0 = pl.multiple_of(j * ZROWS, 8)
            pltpu.sync_copy(zrow_v, out_sh.at[pl.ds(r0, ZROWS)])
            pltpu.sync_copy(zden_v, den_sh.at[pl.ds(r0 * HH, ZROWS * HH)])

    plsc.subcore_barrier()

    iota16 = lax.iota(jnp.int32, 16)
    base = s * E_TEC
    hbase = c * HH

    def chunk(ci, carry):
        eb = pl.multiple_of(base + ci * CHUNK, 8)
        pltpu.sync_copy(src_hbm.at[pl.ds(eb, CHUNK)], src_v)
        pltpu.sync_copy(dst_hbm.at[pl.ds(eb, CHUNK)], dst_v)

        @pl.when(c == 0)
        def _():
            pltpu.async_copy(h0_hbm.at[src_v], rows_v, sem_rows)

        @pl.when(c != 0)
        def _():
            pltpu.async_copy(h1_hbm.at[src_v], rows_v, sem_rows)

        for g in range(GROUPS):
            src16 = src_v[pl.ds(g * 16, 16)]
            dst16 = dst_v[pl.ds(g * 16, 16)]
            for hh in range(HH):
                asrc = plsc.load_gather(a_v, [src16 * 4 + hh])
                adst = plsc.load_gather(a_v, [dst16 * 4 + (2 + hh)])
                al = asrc + adst
                al = jnp.where(al >= 0.0, al, 0.2 * al)
                amx = plsc.load_gather(amax_v, [iota16 * 0 + (hbase + hh)])
                bnd = amx + adst
                bnd = jnp.where(bnd >= 0.0, bnd, 0.2 * bnd)
                p16 = jnp.exp(al - bnd)
                pv = p0_v if hh == 0 else p1_v
                div = di0_v if hh == 0 else di1_v
                pv[pl.ds(16 + g * 16, 16)] = p16
                div[pl.ds(g * 16, 16)] = dst16 * HH + hh
        pltpu.sync_copy(p0_v.at[pl.ds(16, CHUNK)], den_sh.at[di0_v], add=True)
        pltpu.sync_copy(p1_v.at[pl.ds(16, CHUNK)], den_sh.at[di1_v], add=True)
        pltpu.make_async_copy(h0_hbm.at[src_v], rows_v, sem_rows).wait()
        for g in range(GROUPS):
            for hh in range(HH):
                pv = p0_v if hh == 0 else p1_v
                for j in range(16):
                    e = g * 16 + j
                    w = plsc.load_gather(pv, [iota16 * 0 + (16 + e)])
                    rows_v[e, pl.ds(hh * C, 16)] = rows_v[e, pl.ds(hh * C, 16)] * w
                    rows_v[e, pl.ds(hh * C + 16, 16)] = (
                        rows_v[e, pl.ds(hh * C + 16, 16)] * w)
        pltpu.sync_copy(rows_v, out_sh.at[dst_v], add=True)
        return carry

    lax.fori_loop(0, NCHUNK, chunk, 0)
    plsc.subcore_barrier()
    for i in range(WB_TEC):
        j = s * WB_TEC + i

        @pl.when(j < NWB)
        def _():
            r0 = pl.multiple_of(j * ZROWS, 8)
            pltpu.sync_copy(out_sh.at[pl.ds(r0, ZROWS)], zrow_v)
            pltpu.sync_copy(zrow_v, out_hbm.at[c, pl.ds(r0, ZROWS)])
            pltpu.sync_copy(den_sh.at[pl.ds(r0 * HH, ZROWS * HH)], zden_v)
            d0 = pl.multiple_of(c * (N * HH) + r0 * HH, 8)
            pltpu.sync_copy(zden_v, den_hbm.at[pl.ds(d0, ZROWS * HH)])


@functools.lru_cache(maxsize=1)
def _get_sc_call():
    mesh = plsc.VectorSubcoreMesh(
        core_axis_name="c", subcore_axis_name="s",
        num_cores=NC, num_subcores=NS)
    return pl.kernel(
        _sc_body,
        out_type=[
            jax.ShapeDtypeStruct((NC, N, HW), jnp.float32),
            jax.ShapeDtypeStruct((NC * N * HH,), jnp.float32),
        ],
        mesh=mesh,
        compiler_params=pltpu.CompilerParams(
            needs_layout_passes=False, use_tc_tiling_on_sc=False),
        scratch_types=[
            pltpu.VMEM((N * 4,), jnp.float32),
            pltpu.VMEM((CHUNK,), jnp.int32),
            pltpu.VMEM((CHUNK,), jnp.int32),
            pltpu.VMEM((CHUNK, HW), jnp.float32),
            pltpu.VMEM((16 + CHUNK,), jnp.float32),
            pltpu.VMEM((16 + CHUNK,), jnp.float32),
            pltpu.VMEM((CHUNK,), jnp.int32),
            pltpu.VMEM((CHUNK,), jnp.int32),
            pltpu.VMEM((16,), jnp.float32),
            pltpu.VMEM((ZROWS, HW), jnp.float32),
            pltpu.VMEM((ZROWS * HH,), jnp.float32),
            pltpu.VMEM_SHARED((N, HW), jnp.float32),
            pltpu.VMEM_SHARED((N * HH,), jnp.float32),
            pltpu.SemaphoreType.DMA,
            pltpu.SemaphoreType.DMA,
            pltpu.SemaphoreType.DMA,
        ],
    )


def _fin_body(acc_ref, den_ref, bias_ref, cnt_ref, b2_ref, exp_ref, out_ref):
    inv = lax.rsqrt(jnp.maximum(cnt_ref[...], 1.0))
    b = b2_ref[...]
    onehot = (b == lax.broadcasted_iota(jnp.int32, (BLK, NG), 1)).astype(jnp.float32)
    gamma = jnp.sum(onehot * inv, axis=1, keepdims=True)
    for c in range(NC):
        acc = acc_ref[c]
        den = den_ref[c] + 1e-16
        denx = jnp.dot(den, exp_ref[...], preferred_element_type=jnp.float32)
        o = acc / denx + bias_ref[:, c * HW:(c + 1) * HW]
        o = jnp.maximum(o, 0.0)
        out_ref[:, c * HW:(c + 1) * HW] = o * gamma


_fin_call = pl.pallas_call(
    _fin_body,
    grid=(NBLK,),
    in_specs=[
        pl.BlockSpec((NC, BLK, HW), lambda i: (0, i, 0)),
        pl.BlockSpec((NC, BLK, HH), lambda i: (0, i, 0)),
        pl.BlockSpec((1, HC), lambda i: (0, 0)),
        pl.BlockSpec((1, NG), lambda i: (0, 0)),
        pl.BlockSpec((BLK, 1), lambda i: (i, 0)),
        pl.BlockSpec((HH, HW), lambda i: (0, 0)),
    ],
    out_specs=pl.BlockSpec((BLK, HC), lambda i: (i, 0)),
    out_shape=jax.ShapeDtypeStruct((N, HC), jnp.float32),
)


def kernel(x, edge_index, batch, W, att_src, att_dst, bias):
    eyeH = jnp.eye(H, dtype=jnp.float32)
    a_s = (att_src[:, :, None] * eyeH[:, None, :]).reshape(HC, H)
    a_d = (att_dst[:, :, None] * eyeH[:, None, :]).reshape(HC, H)
    a2 = jnp.concatenate([a_s, a_d], axis=1)
    batch2 = batch.reshape(N, 1)

    h0, h1, a, amax, counts = _proj_call(x, W, a2, batch2)

    a0 = a[:, (0, 1, H, H + 1)].reshape(N * 4)
    a1 = a[:, (2, 3, H + 2, H + 3)].reshape(N * 4)
    src = edge_index[0]
    dst = edge_index[1]
    zrow = jnp.zeros((ZROWS, HW), jnp.float32)
    zden = jnp.zeros((ZROWS * HH,), jnp.float32)
    out2, den2 = _get_sc_call()(
        h0, h1, a0, a1, amax.reshape(16), src, dst, zrow, zden)

    expand = jnp.kron(jnp.eye(HH, dtype=jnp.float32), jnp.ones((1, C), jnp.float32))
    return _fin_call(out2, den2.reshape(NC, N, HH), bias.reshape(1, HC),
                     counts, batch2, expand)
